# baseline (device time: 182389 ns/iter reference)
import jax
import jax.numpy as jnp
from jax import lax
from jax.experimental import pallas as pl
from jax.experimental.pallas import tpu as pltpu

N_DEV = 16
SQ = 1024
SKV = 1024
H_LOCAL = 8
DH = 128
D_MODEL = 1024
CHUNK = SQ // N_DEV
SCALE = 0.08838834764831843


def kernel(x, Wq, K_ext, V_ext, Wo):
    def body(x_ref, wq_ref, k_ref, v_ref, wo_ref, out_ref,
             wq_s, wo_s, k_s, v_s, ctx_s,
             send_cw, recv_cw, send_ccw, recv_ccw,
             ssem_cw, rsem_cw, ssem_ccw, rsem_ccw, load_sems):
        my = lax.axis_index("i")
        left = lax.rem(my + N_DEV - 1, N_DEV)
        right = lax.rem(my + 1, N_DEV)

        cp_wq = pltpu.make_async_copy(
            wq_ref.at[:, pl.ds(my * D_MODEL, D_MODEL)], wq_s, load_sems.at[0])
        cp_wo = pltpu.make_async_copy(
            wo_ref.at[pl.ds(my * D_MODEL, D_MODEL), :], wo_s, load_sems.at[1])
        cp_wq.start()
        cp_wo.start()

        def kv_copy(h):
            s = h % 2
            ck = pltpu.make_async_copy(
                k_ref.at[0, :, h, :], k_s.at[s], load_sems.at[2 + 2 * s])
            cv = pltpu.make_async_copy(
                v_ref.at[0, :, h, :], v_s.at[s], load_sems.at[3 + 2 * s])
            return ck, cv

        cp_kv = kv_copy(0)
        cp_kv[0].start()
        cp_kv[1].start()

        barrier_sem = pltpu.get_barrier_semaphore()
        for nbr in (left, right):
            pl.semaphore_signal(
                barrier_sem, inc=1,
                device_id=(nbr,), device_id_type=pl.DeviceIdType.MESH,
            )
        pl.semaphore_wait(barrier_sem, 2)

        cp_wq.wait()
        xb = x_ref[0].astype(jnp.bfloat16)
        wqb = wq_s[...].astype(jnp.bfloat16)
        q_all = jnp.dot(xb, wqb, preferred_element_type=jnp.float32)

        qb = lax.broadcasted_iota(jnp.int32, (SQ, SKV), 0) // 64
        kb = lax.broadcasted_iota(jnp.int32, (SQ, SKV), 1) // 64
        mask = (qb == kb) | (kb == 0) | (lax.rem(qb + kb, 3) == 0)
        bias = jnp.where(mask, 0.0, -1e9).astype(jnp.float32)

        for h in range(H_LOCAL):
            ck, cv = cp_kv
            ck.wait()
            cv.wait()
            if h + 1 < H_LOCAL:
                cp_kv = kv_copy(h + 1)
                cp_kv[0].start()
                cp_kv[1].start()
            q_h = q_all[:, h * DH:(h + 1) * DH].astype(jnp.bfloat16)
            k_h = k_s[h % 2].astype(jnp.bfloat16)
            v_h = v_s[h % 2].astype(jnp.bfloat16)
            scores = lax.dot_general(
                q_h, k_h, (((1,), (1,)), ((), ())),
                preferred_element_type=jnp.float32,
            ) * SCALE + bias
            e = jnp.exp(scores)
            recip = 1.0 / jnp.sum(e, axis=1, keepdims=True)
            w = e * recip
            ctx_h = jnp.dot(w.astype(jnp.bfloat16), v_h,
                            preferred_element_type=jnp.float32)
            ctx_s[:, h * DH:(h + 1) * DH] = ctx_h.astype(jnp.bfloat16)

        cp_wo.wait()
        wob = wo_s[...].astype(jnp.bfloat16)
        out_ref[0] = jnp.dot(ctx_s[...], wob,
                             preferred_element_type=jnp.float32)

        HALF = D_MODEL // 2

        def hop(slot):
            r_cw = pltpu.make_async_remote_copy(
                src_ref=send_cw.at[slot], dst_ref=recv_cw.at[slot],
                send_sem=ssem_cw.at[slot], recv_sem=rsem_cw.at[slot],
                device_id=(right,), device_id_type=pl.DeviceIdType.MESH,
            )
            r_ccw = pltpu.make_async_remote_copy(
                src_ref=send_ccw.at[slot], dst_ref=recv_ccw.at[slot],
                send_sem=ssem_ccw.at[slot], recv_sem=rsem_ccw.at[slot],
                device_id=(left,), device_id_type=pl.DeviceIdType.MESH,
            )
            r_cw.start()
            r_ccw.start()
            r_cw.wait()
            r_ccw.wait()

        def rows(c):
            return pl.ds(c * CHUNK, CHUNK)

        send_cw[0] = out_ref[0, rows(my), :HALF]
        send_ccw[0] = out_ref[0, rows(my), HALF:]
        for s in range(N_DEV - 1):
            slot = s % 2
            hop(slot)
            c_cw = lax.rem(my + (2 * N_DEV - 1 - s), N_DEV)
            c_ccw = lax.rem(my + 1 + s, N_DEV)
            v_cw = recv_cw[slot] + out_ref[0, rows(c_cw), :HALF]
            v_ccw = recv_ccw[slot] + out_ref[0, rows(c_ccw), HALF:]
            if s < N_DEV - 2:
                send_cw[(s + 1) % 2] = v_cw
                send_ccw[(s + 1) % 2] = v_ccw
            else:
                out_ref[0, rows(lax.rem(my + 1, N_DEV)), :HALF] = v_cw
                out_ref[0, rows(lax.rem(my + N_DEV - 1, N_DEV)), HALF:] = v_ccw
                send_cw[(s + 1) % 2] = v_cw
                send_ccw[(s + 1) % 2] = v_ccw

        for g in range(N_DEV - 1):
            slot = (N_DEV - 1 + g) % 2
            hop(slot)
            c_cw = lax.rem(my + (N_DEV - g), N_DEV)
            c_ccw = lax.rem(my + g, N_DEV)
            out_ref[0, rows(c_cw), :HALF] = recv_cw[slot]
            out_ref[0, rows(c_ccw), HALF:] = recv_ccw[slot]
            if g < N_DEV - 2:
                send_cw[(g + N_DEV) % 2] = recv_cw[slot]
                send_ccw[(g + N_DEV) % 2] = recv_ccw[slot]

    return pl.pallas_call(
        body,
        out_shape=jax.ShapeDtypeStruct((1, SQ, D_MODEL), jnp.float32),
        in_specs=[
            pl.BlockSpec(memory_space=pltpu.VMEM),
            pl.BlockSpec(memory_space=pl.ANY),
            pl.BlockSpec(memory_space=pl.ANY),
            pl.BlockSpec(memory_space=pl.ANY),
            pl.BlockSpec(memory_space=pl.ANY),
        ],
        out_specs=pl.BlockSpec(memory_space=pltpu.VMEM),
        scratch_shapes=[
            pltpu.VMEM((D_MODEL, D_MODEL), jnp.float32),
            pltpu.VMEM((D_MODEL, D_MODEL), jnp.float32),
            pltpu.VMEM((2, SKV, DH), jnp.float32),
            pltpu.VMEM((2, SKV, DH), jnp.float32),
            pltpu.VMEM((SQ, D_MODEL), jnp.bfloat16),
            pltpu.VMEM((2, CHUNK, D_MODEL // 2), jnp.float32),
            pltpu.VMEM((2, CHUNK, D_MODEL // 2), jnp.float32),
            pltpu.VMEM((2, CHUNK, D_MODEL // 2), jnp.float32),
            pltpu.VMEM((2, CHUNK, D_MODEL // 2), jnp.float32),
            pltpu.SemaphoreType.DMA((2,)),
            pltpu.SemaphoreType.DMA((2,)),
            pltpu.SemaphoreType.DMA((2,)),
            pltpu.SemaphoreType.DMA((2,)),
            pltpu.SemaphoreType.DMA((6,)),
        ],
        compiler_params=pltpu.CompilerParams(collective_id=0),
    )(x, Wq, K_ext, V_ext, Wo)


# device time: 120942 ns/iter; 1.5081x vs baseline; 1.5081x over previous
import os

import jax
import jax.numpy as jnp
from jax import lax
from jax.experimental import pallas as pl
from jax.experimental.pallas import tpu as pltpu

_ABLATE = os.environ.get('KERNEL_ABLATE', '')

N_DEV = 16
SQ = 1024
SKV = 1024
H_LOCAL = 8
DH = 128
D_MODEL = 1024
CHUNK = SQ // N_DEV
SCALE = 0.08838834764831843

RING = (0, 4, 8, 12, 13, 9, 5, 6, 10, 14, 15, 11, 7, 3, 2, 1)
INV_RING = tuple(RING.index(i) for i in range(N_DEV))


def kernel(x, Wq, K_ext, V_ext, Wo):
    def body(ring_ref, inv_ref, x_ref, wq_ref, k_ref, v_ref, wo_ref, out_ref,
             wq_s, wo_s, k_s, v_s, ctx_s,
             send_cw, recv_cw, send_ccw, recv_ccw,
             ssem_cw, rsem_cw, ssem_ccw, rsem_ccw, load_sems):
        my = lax.axis_index("i")
        r = inv_ref[my]
        right = ring_ref[lax.rem(r + 1, N_DEV)]
        left = ring_ref[lax.rem(r + N_DEV - 1, N_DEV)]

        def run_ring():
            HALF = D_MODEL // 2

            def hop(slot):
                r_cw = pltpu.make_async_remote_copy(
                    src_ref=send_cw.at[slot], dst_ref=recv_cw.at[slot],
                    send_sem=ssem_cw.at[slot], recv_sem=rsem_cw.at[slot],
                    device_id=(right,), device_id_type=pl.DeviceIdType.MESH,
                )
                r_ccw = pltpu.make_async_remote_copy(
                    src_ref=send_ccw.at[slot], dst_ref=recv_ccw.at[slot],
                    send_sem=ssem_ccw.at[slot], recv_sem=rsem_ccw.at[slot],
                    device_id=(left,), device_id_type=pl.DeviceIdType.MESH,
                )
                r_cw.start()
                r_ccw.start()
                r_cw.wait()
                r_ccw.wait()

            def rows(c):
                return pl.ds(c * CHUNK, CHUNK)

            send_cw[0] = out_ref[0, rows(r), :HALF].astype(jnp.bfloat16)
            send_ccw[0] = out_ref[0, rows(r), HALF:].astype(jnp.bfloat16)
            for s in range(N_DEV - 1):
                slot = s % 2
                hop(slot)
                c_cw = lax.rem(r + (2 * N_DEV - 1 - s), N_DEV)
                c_ccw = lax.rem(r + 1 + s, N_DEV)
                v_cw = (recv_cw[slot].astype(jnp.float32)
                        + out_ref[0, rows(c_cw), :HALF])
                v_ccw = (recv_ccw[slot].astype(jnp.float32)
                         + out_ref[0, rows(c_ccw), HALF:])
                if s < N_DEV - 2:
                    send_cw[(s + 1) % 2] = v_cw.astype(jnp.bfloat16)
                    send_ccw[(s + 1) % 2] = v_ccw.astype(jnp.bfloat16)
                else:
                    out_ref[0, rows(lax.rem(r + 1, N_DEV)), :HALF] = v_cw
                    out_ref[0, rows(lax.rem(r + N_DEV - 1, N_DEV)),
                            HALF:] = v_ccw
                    send_cw[(s + 1) % 2] = v_cw.astype(jnp.bfloat16)
                    send_ccw[(s + 1) % 2] = v_ccw.astype(jnp.bfloat16)

            for g in range(N_DEV - 1):
                slot = (N_DEV - 1 + g) % 2
                hop(slot)
                c_cw = lax.rem(r + (N_DEV - g), N_DEV)
                c_ccw = lax.rem(r + g, N_DEV)
                out_ref[0, rows(c_cw), :HALF] = \
                    recv_cw[slot].astype(jnp.float32)
                out_ref[0, rows(c_ccw), HALF:] = \
                    recv_ccw[slot].astype(jnp.float32)
                if g < N_DEV - 2:
                    send_cw[(g + N_DEV) % 2] = recv_cw[slot]
                    send_ccw[(g + N_DEV) % 2] = recv_ccw[slot]

        cp_wq = pltpu.make_async_copy(
            wq_ref.at[:, pl.ds(my * D_MODEL, D_MODEL)], wq_s, load_sems.at[0])
        cp_wo = pltpu.make_async_copy(
            wo_ref.at[pl.ds(my * D_MODEL, D_MODEL), :], wo_s, load_sems.at[1])
        cp_wq.start()
        cp_wo.start()

        def kv_copy(h):
            s = h % 2
            ck = pltpu.make_async_copy(
                k_ref.at[0, :, h, :], k_s.at[s], load_sems.at[2 + 2 * s])
            cv = pltpu.make_async_copy(
                v_ref.at[0, :, h, :], v_s.at[s], load_sems.at[3 + 2 * s])
            return ck, cv

        cp_kv = kv_copy(0)
        cp_kv[0].start()
        cp_kv[1].start()

        barrier_sem = pltpu.get_barrier_semaphore()
        for nbr in (left, right):
            pl.semaphore_signal(
                barrier_sem, inc=1,
                device_id=(nbr,), device_id_type=pl.DeviceIdType.MESH,
            )
        pl.semaphore_wait(barrier_sem, 2)

        if _ABLATE == 'nocomp':
            cp_wq.wait()
            cp_wo.wait()
            cp_kv[0].wait()
            cp_kv[1].wait()
            out_ref[0] = x_ref[0]
            run_ring()
            return
        cp_wq.wait()
        xb = x_ref[0].astype(jnp.bfloat16)
        wqb = wq_s[...].astype(jnp.bfloat16)
        q_all = jnp.dot(xb, wqb, preferred_element_type=jnp.float32)

        qb = lax.broadcasted_iota(jnp.int32, (SQ, SKV), 0) // 64
        kb = lax.broadcasted_iota(jnp.int32, (SQ, SKV), 1) // 64
        mask = (qb == kb) | (kb == 0) | (lax.rem(qb + kb, 3) == 0)
        bias = jnp.where(mask, 0.0, -1e9).astype(jnp.float32)

        for h in range(H_LOCAL):
            ck, cv = cp_kv
            ck.wait()
            cv.wait()
            if h + 1 < H_LOCAL:
                cp_kv = kv_copy(h + 1)
                cp_kv[0].start()
                cp_kv[1].start()
            q_h = q_all[:, h * DH:(h + 1) * DH].astype(jnp.bfloat16)
            k_h = k_s[h % 2].astype(jnp.bfloat16)
            v_h = v_s[h % 2].astype(jnp.bfloat16)
            scores = lax.dot_general(
                q_h, k_h, (((1,), (1,)), ((), ())),
                preferred_element_type=jnp.float32,
            ) * SCALE + bias
            e = jnp.exp(scores)
            recip = 1.0 / jnp.sum(e, axis=1, keepdims=True)
            w = e * recip
            ctx_h = jnp.dot(w.astype(jnp.bfloat16), v_h,
                            preferred_element_type=jnp.float32)
            ctx_s[:, h * DH:(h + 1) * DH] = ctx_h.astype(jnp.bfloat16)

        cp_wo.wait()
        wob = wo_s[...].astype(jnp.bfloat16)
        out_ref[0] = jnp.dot(ctx_s[...], wob,
                             preferred_element_type=jnp.float32)

        if _ABLATE != 'noring':
            run_ring()

    return pl.pallas_call(
        body,
        out_shape=jax.ShapeDtypeStruct((1, SQ, D_MODEL), jnp.float32),
        in_specs=[
            pl.BlockSpec(memory_space=pltpu.SMEM),
            pl.BlockSpec(memory_space=pltpu.SMEM),
            pl.BlockSpec(memory_space=pltpu.VMEM),
            pl.BlockSpec(memory_space=pl.ANY),
            pl.BlockSpec(memory_space=pl.ANY),
            pl.BlockSpec(memory_space=pl.ANY),
            pl.BlockSpec(memory_space=pl.ANY),
        ],
        out_specs=pl.BlockSpec(memory_space=pltpu.VMEM),
        scratch_shapes=[
            pltpu.VMEM((D_MODEL, D_MODEL), jnp.float32),
            pltpu.VMEM((D_MODEL, D_MODEL), jnp.float32),
            pltpu.VMEM((2, SKV, DH), jnp.float32),
            pltpu.VMEM((2, SKV, DH), jnp.float32),
            pltpu.VMEM((SQ, D_MODEL), jnp.bfloat16),
            pltpu.VMEM((2, CHUNK, D_MODEL // 2), jnp.bfloat16),
            pltpu.VMEM((2, CHUNK, D_MODEL // 2), jnp.bfloat16),
            pltpu.VMEM((2, CHUNK, D_MODEL // 2), jnp.bfloat16),
            pltpu.VMEM((2, CHUNK, D_MODEL // 2), jnp.bfloat16),
            pltpu.SemaphoreType.DMA((2,)),
            pltpu.SemaphoreType.DMA((2,)),
            pltpu.SemaphoreType.DMA((2,)),
            pltpu.SemaphoreType.DMA((2,)),
            pltpu.SemaphoreType.DMA((6,)),
        ],
        compiler_params=pltpu.CompilerParams(collective_id=0),
    )(jnp.array(RING, dtype=jnp.int32), jnp.array(INV_RING, dtype=jnp.int32),
      x, Wq, K_ext, V_ext, Wo)


# device time: 118776 ns/iter; 1.5356x vs baseline; 1.0182x over previous
import os

import jax
import jax.numpy as jnp
from jax import lax
from jax.experimental import pallas as pl
from jax.experimental.pallas import tpu as pltpu

_ABLATE = os.environ.get('KERNEL_ABLATE', '')

N_DEV = 16
SQ = 1024
SKV = 1024
H_LOCAL = 8
DH = 128
D_MODEL = 1024
CHUNK = SQ // N_DEV
SCALE = 0.08838834764831843

RING = (0, 4, 8, 12, 13, 9, 5, 6, 10, 14, 15, 11, 7, 3, 2, 1)
INV_RING = tuple(RING.index(i) for i in range(N_DEV))


def kernel(x, Wq, K_ext, V_ext, Wo):
    def body(ring_ref, inv_ref, x_ref, wq_ref, k_ref, v_ref, wo_ref, out_ref,
             wq_s, wo_s, k_s, v_s, ctx_s,
             send_cw, recv_cw, send_ccw, recv_ccw,
             ssem_cw, rsem_cw, ssem_ccw, rsem_ccw, load_sems):
        my = lax.axis_index("i")
        r = inv_ref[my]
        right = ring_ref[lax.rem(r + 1, N_DEV)]
        left = ring_ref[lax.rem(r + N_DEV - 1, N_DEV)]

        def run_ring(partial_chunk=None):
            HALF = D_MODEL // 2

            def hop_start(slot):
                r_cw = pltpu.make_async_remote_copy(
                    src_ref=send_cw.at[slot], dst_ref=recv_cw.at[slot],
                    send_sem=ssem_cw.at[slot], recv_sem=rsem_cw.at[slot],
                    device_id=(right,), device_id_type=pl.DeviceIdType.MESH,
                )
                r_ccw = pltpu.make_async_remote_copy(
                    src_ref=send_ccw.at[slot], dst_ref=recv_ccw.at[slot],
                    send_sem=ssem_ccw.at[slot], recv_sem=rsem_ccw.at[slot],
                    device_id=(left,), device_id_type=pl.DeviceIdType.MESH,
                )
                r_cw.start()
                r_ccw.start()
                return r_cw, r_ccw

            def hop(slot):
                r_cw, r_ccw = hop_start(slot)
                r_cw.wait()
                r_ccw.wait()

            def rows(c):
                return pl.ds(c * CHUNK, CHUNK)

            if partial_chunk is None:
                def partial_chunk(c, lo, hi):
                    return out_ref[0, rows(c), lo:hi]
            seed = partial_chunk(r, 0, D_MODEL)
            send_cw[0] = seed[:, :HALF].astype(jnp.bfloat16)
            send_ccw[0] = seed[:, HALF:].astype(jnp.bfloat16)
            for s in range(N_DEV - 1):
                slot = s % 2
                c_cw = lax.rem(r + (2 * N_DEV - 1 - s), N_DEV)
                c_ccw = lax.rem(r + 1 + s, N_DEV)
                r_cw, r_ccw = hop_start(slot)
                p_cw = partial_chunk(c_cw, 0, HALF)
                p_ccw = partial_chunk(c_ccw, HALF, D_MODEL)
                r_cw.wait()
                r_ccw.wait()
                v_cw = recv_cw[slot].astype(jnp.float32) + p_cw
                v_ccw = recv_ccw[slot].astype(jnp.float32) + p_ccw
                if s < N_DEV - 2:
                    send_cw[(s + 1) % 2] = v_cw.astype(jnp.bfloat16)
                    send_ccw[(s + 1) % 2] = v_ccw.astype(jnp.bfloat16)
                else:
                    out_ref[0, rows(lax.rem(r + 1, N_DEV)), :HALF] = v_cw
                    out_ref[0, rows(lax.rem(r + N_DEV - 1, N_DEV)),
                            HALF:] = v_ccw
                    send_cw[(s + 1) % 2] = v_cw.astype(jnp.bfloat16)
                    send_ccw[(s + 1) % 2] = v_ccw.astype(jnp.bfloat16)

            for g in range(N_DEV - 1):
                slot = (N_DEV - 1 + g) % 2
                hop(slot)
                c_cw = lax.rem(r + (N_DEV - g), N_DEV)
                c_ccw = lax.rem(r + g, N_DEV)
                out_ref[0, rows(c_cw), :HALF] = \
                    recv_cw[slot].astype(jnp.float32)
                out_ref[0, rows(c_ccw), HALF:] = \
                    recv_ccw[slot].astype(jnp.float32)
                if g < N_DEV - 2:
                    send_cw[(g + N_DEV) % 2] = recv_cw[slot]
                    send_ccw[(g + N_DEV) % 2] = recv_ccw[slot]

        cp_wq = pltpu.make_async_copy(
            wq_ref.at[:, pl.ds(my * D_MODEL, D_MODEL)], wq_s, load_sems.at[0])
        cp_wo = pltpu.make_async_copy(
            wo_ref.at[pl.ds(my * D_MODEL, D_MODEL), :], wo_s, load_sems.at[1])
        cp_wq.start()
        cp_wo.start()

        def kv_copy(h):
            s = h % 2
            ck = pltpu.make_async_copy(
                k_ref.at[0, :, h, :], k_s.at[s], load_sems.at[2 + 2 * s])
            cv = pltpu.make_async_copy(
                v_ref.at[0, :, h, :], v_s.at[s], load_sems.at[3 + 2 * s])
            return ck, cv

        cp_kv = kv_copy(0)
        cp_kv[0].start()
        cp_kv[1].start()

        barrier_sem = pltpu.get_barrier_semaphore()
        for nbr in (left, right):
            pl.semaphore_signal(
                barrier_sem, inc=1,
                device_id=(nbr,), device_id_type=pl.DeviceIdType.MESH,
            )
        pl.semaphore_wait(barrier_sem, 2)

        if _ABLATE == 'nocomp':
            cp_wq.wait()
            cp_wo.wait()
            cp_kv[0].wait()
            cp_kv[1].wait()
            out_ref[0] = x_ref[0]
            run_ring()
            return
        cp_wq.wait()
        xb = x_ref[0].astype(jnp.bfloat16)
        wqb = wq_s[...].astype(jnp.bfloat16)
        q_all = jnp.dot(xb, wqb, preferred_element_type=jnp.float32)

        qb = lax.broadcasted_iota(jnp.int32, (SQ, SKV), 0) // 64
        kb = lax.broadcasted_iota(jnp.int32, (SQ, SKV), 1) // 64
        mask = (qb == kb) | (kb == 0) | (lax.rem(qb + kb, 3) == 0)
        bias = jnp.where(mask, 0.0, -1e9).astype(jnp.float32)

        for h in range(H_LOCAL):
            ck, cv = cp_kv
            ck.wait()
            cv.wait()
            if h + 1 < H_LOCAL:
                cp_kv = kv_copy(h + 1)
                cp_kv[0].start()
                cp_kv[1].start()
            q_h = q_all[:, h * DH:(h + 1) * DH].astype(jnp.bfloat16)
            k_h = k_s[h % 2].astype(jnp.bfloat16)
            v_h = v_s[h % 2].astype(jnp.bfloat16)
            scores = lax.dot_general(
                q_h, k_h, (((1,), (1,)), ((), ())),
                preferred_element_type=jnp.float32,
            ) * SCALE + bias
            e = jnp.exp(scores)
            recip = 1.0 / jnp.sum(e, axis=1, keepdims=True)
            w = e * recip
            ctx_h = jnp.dot(w.astype(jnp.bfloat16), v_h,
                            preferred_element_type=jnp.float32)
            ctx_s[:, h * DH:(h + 1) * DH] = ctx_h.astype(jnp.bfloat16)

        cp_wo.wait()
        wob = wo_s[...].astype(jnp.bfloat16)

        def partial_chunk(c, lo, hi):
            return jnp.dot(ctx_s[pl.ds(c * CHUNK, CHUNK), :], wob[:, lo:hi],
                           preferred_element_type=jnp.float32)

        if _ABLATE == 'noring':
            out_ref[0] = jnp.dot(ctx_s[...], wob,
                                 preferred_element_type=jnp.float32)
        else:
            run_ring(partial_chunk)

    return pl.pallas_call(
        body,
        out_shape=jax.ShapeDtypeStruct((1, SQ, D_MODEL), jnp.float32),
        in_specs=[
            pl.BlockSpec(memory_space=pltpu.SMEM),
            pl.BlockSpec(memory_space=pltpu.SMEM),
            pl.BlockSpec(memory_space=pltpu.VMEM),
            pl.BlockSpec(memory_space=pl.ANY),
            pl.BlockSpec(memory_space=pl.ANY),
            pl.BlockSpec(memory_space=pl.ANY),
            pl.BlockSpec(memory_space=pl.ANY),
        ],
        out_specs=pl.BlockSpec(memory_space=pltpu.VMEM),
        scratch_shapes=[
            pltpu.VMEM((D_MODEL, D_MODEL), jnp.float32),
            pltpu.VMEM((D_MODEL, D_MODEL), jnp.float32),
            pltpu.VMEM((2, SKV, DH), jnp.float32),
            pltpu.VMEM((2, SKV, DH), jnp.float32),
            pltpu.VMEM((SQ, D_MODEL), jnp.bfloat16),
            pltpu.VMEM((2, CHUNK, D_MODEL // 2), jnp.bfloat16),
            pltpu.VMEM((2, CHUNK, D_MODEL // 2), jnp.bfloat16),
            pltpu.VMEM((2, CHUNK, D_MODEL // 2), jnp.bfloat16),
            pltpu.VMEM((2, CHUNK, D_MODEL // 2), jnp.bfloat16),
            pltpu.SemaphoreType.DMA((2,)),
            pltpu.SemaphoreType.DMA((2,)),
            pltpu.SemaphoreType.DMA((2,)),
            pltpu.SemaphoreType.DMA((2,)),
            pltpu.SemaphoreType.DMA((6,)),
        ],
        compiler_params=pltpu.CompilerParams(collective_id=0),
    )(jnp.array(RING, dtype=jnp.int32), jnp.array(INV_RING, dtype=jnp.int32),
      x, Wq, K_ext, V_ext, Wo)


# device time: 96824 ns/iter; 1.8837x vs baseline; 1.2267x over previous
import os

import jax
import jax.numpy as jnp
from jax import lax
from jax.experimental import pallas as pl
from jax.experimental.pallas import tpu as pltpu

_ABLATE = os.environ.get('KERNEL_ABLATE', '')

N_DEV = 16
SQ = 1024
SKV = 1024
H_LOCAL = 8
DH = 128
D_MODEL = 1024
HALF = D_MODEL // 2
CHUNK = 64
SUP = 256
SCALE = 0.08838834764831843

GRAY = (0, 1, 3, 2)


def kernel(x, Wq, K_ext, V_ext, Wo):
    def body(gray_ref, x_ref, wq_ref, k_ref, v_ref, wo_ref, out_ref,
             wq_s, wo_s, k_s, v_s, ctx_s,
             sa_cw, ra_cw, sa_ccw, ra_ccw,
             sb_cw, rb_cw, sb_ccw, rb_ccw,
             sc_cw, rc_cw, sc_ccw, rc_ccw,
             semsA, semsB, semsC, load_sems):
        my = lax.axis_index("i")
        q = lax.rem(my, 4)
        z = my // 4
        t = gray_ref[z]
        zr = gray_ref[lax.rem(t + 1, 4)]
        zl = gray_ref[lax.rem(t + 3, 4)]
        right_xy = 4 * z + lax.rem(q + 1, 4)
        left_xy = 4 * z + lax.rem(q + 3, 4)
        right_z = 4 * zr + q
        left_z = 4 * zl + q

        def exchange(slot, sems, s_cw, r_cw, dst_cw, s_ccw, r_ccw, dst_ccw):
            r1 = pltpu.make_async_remote_copy(
                src_ref=s_cw.at[slot], dst_ref=r_cw.at[slot],
                send_sem=sems.at[0, slot], recv_sem=sems.at[1, slot],
                device_id=(dst_cw,), device_id_type=pl.DeviceIdType.MESH)
            r2 = pltpu.make_async_remote_copy(
                src_ref=s_ccw.at[slot], dst_ref=r_ccw.at[slot],
                send_sem=sems.at[2, slot], recv_sem=sems.at[3, slot],
                device_id=(dst_ccw,), device_id_type=pl.DeviceIdType.MESH)
            r1.start()
            r2.start()
            r1.wait()
            r2.wait()

        def sup_rows(j):
            return pl.ds(j * SUP, SUP)

        def run_allreduce():
            bf16 = jnp.bfloat16
            f32 = jnp.float32

            sa_cw[0] = out_ref[0, sup_rows(q), :HALF].astype(bf16)
            sa_ccw[0] = out_ref[0, sup_rows(q), HALF:].astype(bf16)
            for s in range(3):
                slot = s % 2
                exchange(slot, semsA, sa_cw, ra_cw, right_xy,
                         sa_ccw, ra_ccw, left_xy)
                j_cw = lax.rem(q + 7 - s, 4)
                j_ccw = lax.rem(q + 1 + s, 4)
                v_cw = (ra_cw[slot].astype(f32)
                        + out_ref[0, sup_rows(j_cw), :HALF])
                v_ccw = (ra_ccw[slot].astype(f32)
                         + out_ref[0, sup_rows(j_ccw), HALF:])
                if s < 2:
                    sa_cw[(s + 1) % 2] = v_cw.astype(bf16)
                    sa_ccw[(s + 1) % 2] = v_ccw.astype(bf16)
                else:
                    out_ref[0, sup_rows(j_cw), :HALF] = v_cw
                    out_ref[0, sup_rows(j_ccw), HALF:] = v_ccw

            row0_cw = lax.rem(q + 1, 4) * SUP
            row0_ccw = lax.rem(q + 3, 4) * SUP

            def sub_rows(row0, i):
                return pl.ds(row0 + i * CHUNK, CHUNK)

            sb_cw[0] = out_ref[0, sub_rows(row0_cw, t), :HALF].astype(bf16)
            sb_ccw[0] = out_ref[0, sub_rows(row0_ccw, t), HALF:].astype(bf16)
            for s in range(3):
                slot = s % 2
                exchange(slot, semsB, sb_cw, rb_cw, right_z,
                         sb_ccw, rb_ccw, left_z)
                i_cw = lax.rem(t + 7 - s, 4)
                i_ccw = lax.rem(t + 1 + s, 4)
                v_cw = (rb_cw[slot].astype(f32)
                        + out_ref[0, sub_rows(row0_cw, i_cw), :HALF])
                v_ccw = (rb_ccw[slot].astype(f32)
                         + out_ref[0, sub_rows(row0_ccw, i_ccw), HALF:])
                if s < 2:
                    sb_cw[(s + 1) % 2] = v_cw.astype(bf16)
                    sb_ccw[(s + 1) % 2] = v_ccw.astype(bf16)
                else:
                    out_ref[0, sub_rows(row0_cw, i_cw), :HALF] = v_cw
                    out_ref[0, sub_rows(row0_ccw, i_ccw), HALF:] = v_ccw
                    sb_cw[(s + 1) % 2] = v_cw.astype(bf16)
                    sb_ccw[(s + 1) % 2] = v_ccw.astype(bf16)
            for g in range(3):
                slot = (3 + g) % 2
                exchange(slot, semsB, sb_cw, rb_cw, right_z,
                         sb_ccw, rb_ccw, left_z)
                i_cw = lax.rem(t + 4 - g, 4)
                i_ccw = lax.rem(t + g, 4)
                out_ref[0, sub_rows(row0_cw, i_cw), :HALF] = \
                    rb_cw[slot].astype(f32)
                out_ref[0, sub_rows(row0_ccw, i_ccw), HALF:] = \
                    rb_ccw[slot].astype(f32)
                if g < 2:
                    sb_cw[(g + 4) % 2] = rb_cw[slot]
                    sb_ccw[(g + 4) % 2] = rb_ccw[slot]

            sc_cw[0] = out_ref[0, sup_rows(lax.rem(q + 1, 4)),
                               :HALF].astype(bf16)
            sc_ccw[0] = out_ref[0, sup_rows(lax.rem(q + 3, 4)),
                                HALF:].astype(bf16)
            for g in range(3):
                slot = g % 2
                exchange(slot, semsC, sc_cw, rc_cw, right_xy,
                         sc_ccw, rc_ccw, left_xy)
                j_cw = lax.rem(q + 4 - g, 4)
                j_ccw = lax.rem(q + g, 4)
                out_ref[0, sup_rows(j_cw), :HALF] = rc_cw[slot].astype(f32)
                out_ref[0, sup_rows(j_ccw), HALF:] = rc_ccw[slot].astype(f32)
                if g < 2:
                    sc_cw[(g + 1) % 2] = rc_cw[slot]
                    sc_ccw[(g + 1) % 2] = rc_ccw[slot]

        cp_wq = pltpu.make_async_copy(
            wq_ref.at[:, pl.ds(my * D_MODEL, D_MODEL)], wq_s, load_sems.at[0])
        cp_wo = pltpu.make_async_copy(
            wo_ref.at[pl.ds(my * D_MODEL, D_MODEL), :], wo_s, load_sems.at[1])
        cp_wq.start()
        cp_wo.start()

        def kv_copy(h):
            s = h % 2
            ck = pltpu.make_async_copy(
                k_ref.at[0, :, h, :], k_s.at[s], load_sems.at[2 + 2 * s])
            cv = pltpu.make_async_copy(
                v_ref.at[0, :, h, :], v_s.at[s], load_sems.at[3 + 2 * s])
            return ck, cv

        cp_kv = kv_copy(0)
        cp_kv[0].start()
        cp_kv[1].start()

        barrier_sem = pltpu.get_barrier_semaphore()
        for nbr in (left_xy, right_xy, left_z, right_z):
            pl.semaphore_signal(
                barrier_sem, inc=1,
                device_id=(nbr,), device_id_type=pl.DeviceIdType.MESH,
            )
        pl.semaphore_wait(barrier_sem, 4)

        if _ABLATE == 'nocomp':
            cp_wq.wait()
            cp_wo.wait()
            cp_kv[0].wait()
            cp_kv[1].wait()
            out_ref[0] = x_ref[0]
            run_allreduce()
            return
        cp_wq.wait()
        xb = x_ref[0].astype(jnp.bfloat16)
        wqb = wq_s[...].astype(jnp.bfloat16)
        q_all = jnp.dot(xb, wqb, preferred_element_type=jnp.float32)

        qb = lax.broadcasted_iota(jnp.int32, (SQ, SKV), 0) // 64
        kb = lax.broadcasted_iota(jnp.int32, (SQ, SKV), 1) // 64
        mask = (qb == kb) | (kb == 0) | (lax.rem(qb + kb, 3) == 0)
        bias = jnp.where(mask, 0.0, -1e9).astype(jnp.float32)

        for h in range(H_LOCAL):
            ck, cv = cp_kv
            ck.wait()
            cv.wait()
            if h + 1 < H_LOCAL:
                cp_kv = kv_copy(h + 1)
                cp_kv[0].start()
                cp_kv[1].start()
            q_h = q_all[:, h * DH:(h + 1) * DH].astype(jnp.bfloat16)
            k_h = k_s[h % 2].astype(jnp.bfloat16)
            v_h = v_s[h % 2].astype(jnp.bfloat16)
            scores = lax.dot_general(
                q_h, k_h, (((1,), (1,)), ((), ())),
                preferred_element_type=jnp.float32,
            ) * SCALE + bias
            e = jnp.exp(scores)
            recip = 1.0 / jnp.sum(e, axis=1, keepdims=True)
            w = e * recip
            ctx_h = jnp.dot(w.astype(jnp.bfloat16), v_h,
                            preferred_element_type=jnp.float32)
            ctx_s[:, h * DH:(h + 1) * DH] = ctx_h.astype(jnp.bfloat16)

        cp_wo.wait()
        wob = wo_s[...].astype(jnp.bfloat16)
        out_ref[0] = jnp.dot(ctx_s[...], wob,
                             preferred_element_type=jnp.float32)

        if _ABLATE != 'noring':
            run_allreduce()

    return pl.pallas_call(
        body,
        out_shape=jax.ShapeDtypeStruct((1, SQ, D_MODEL), jnp.float32),
        in_specs=[
            pl.BlockSpec(memory_space=pltpu.SMEM),
            pl.BlockSpec(memory_space=pltpu.VMEM),
            pl.BlockSpec(memory_space=pl.ANY),
            pl.BlockSpec(memory_space=pl.ANY),
            pl.BlockSpec(memory_space=pl.ANY),
            pl.BlockSpec(memory_space=pl.ANY),
        ],
        out_specs=pl.BlockSpec(memory_space=pltpu.VMEM),
        scratch_shapes=[
            pltpu.VMEM((D_MODEL, D_MODEL), jnp.float32),
            pltpu.VMEM((D_MODEL, D_MODEL), jnp.float32),
            pltpu.VMEM((2, SKV, DH), jnp.float32),
            pltpu.VMEM((2, SKV, DH), jnp.float32),
            pltpu.VMEM((SQ, D_MODEL), jnp.bfloat16),
            pltpu.VMEM((2, SUP, HALF), jnp.bfloat16),
            pltpu.VMEM((2, SUP, HALF), jnp.bfloat16),
            pltpu.VMEM((2, SUP, HALF), jnp.bfloat16),
            pltpu.VMEM((2, SUP, HALF), jnp.bfloat16),
            pltpu.VMEM((2, CHUNK, HALF), jnp.bfloat16),
            pltpu.VMEM((2, CHUNK, HALF), jnp.bfloat16),
            pltpu.VMEM((2, CHUNK, HALF), jnp.bfloat16),
            pltpu.VMEM((2, CHUNK, HALF), jnp.bfloat16),
            pltpu.VMEM((2, SUP, HALF), jnp.bfloat16),
            pltpu.VMEM((2, SUP, HALF), jnp.bfloat16),
            pltpu.VMEM((2, SUP, HALF), jnp.bfloat16),
            pltpu.VMEM((2, SUP, HALF), jnp.bfloat16),
            pltpu.SemaphoreType.DMA((4, 2)),
            pltpu.SemaphoreType.DMA((4, 2)),
            pltpu.SemaphoreType.DMA((4, 2)),
            pltpu.SemaphoreType.DMA((6,)),
        ],
        compiler_params=pltpu.CompilerParams(collective_id=0),
    )(jnp.array(GRAY, dtype=jnp.int32), x, Wq, K_ext, V_ext, Wo)


# device time: 91721 ns/iter; 1.9885x vs baseline; 1.0556x over previous
import os

import jax
import jax.numpy as jnp
from jax import lax
from jax.experimental import pallas as pl
from jax.experimental.pallas import tpu as pltpu

_ABLATE = os.environ.get('KERNEL_ABLATE', '')

N_DEV = 16
SQ = 1024
SKV = 1024
H_LOCAL = 8
DH = 128
D_MODEL = 1024
HALF = D_MODEL // 2
CHUNK = 64
SUP = 256
SCALE = 0.08838834764831843

GRAY = (0, 1, 3, 2)


def kernel(x, Wq, K_ext, V_ext, Wo):
    def body(x_ref, wq_ref, k_ref, v_ref, wo_ref, out_ref,
             wq_s, wo_s, k_s, v_s, ctx_s,
             sa_cw, ra_cw, sa_ccw, ra_ccw,
             sb_x, rb_x,
             sc_cw, rc_cw, sc_ccw, rc_ccw,
             semsA, semsB, semsC, load_sems):
        my = lax.axis_index("i")
        q = lax.rem(my, 4)
        z = my // 4
        right_xy = 4 * z + lax.rem(q + 1, 4)
        left_xy = 4 * z + lax.rem(q + 3, 4)
        b0 = lax.rem(z, 2)
        b1 = lax.rem(z // 2, 2)
        p1 = 4 * (z + 1 - 2 * b0) + q
        p2 = 4 * (z + 2 - 4 * b1) + q

        def exchange(slot, sems, s_cw, r_cw, dst_cw, s_ccw, r_ccw, dst_ccw):
            r1 = pltpu.make_async_remote_copy(
                src_ref=s_cw.at[slot], dst_ref=r_cw.at[slot],
                send_sem=sems.at[0, slot], recv_sem=sems.at[1, slot],
                device_id=(dst_cw,), device_id_type=pl.DeviceIdType.MESH)
            r2 = pltpu.make_async_remote_copy(
                src_ref=s_ccw.at[slot], dst_ref=r_ccw.at[slot],
                send_sem=sems.at[2, slot], recv_sem=sems.at[3, slot],
                device_id=(dst_ccw,), device_id_type=pl.DeviceIdType.MESH)
            r1.start()
            r2.start()
            r1.wait()
            r2.wait()

        def sup_rows(j):
            return pl.ds(j * SUP, SUP)

        def run_allreduce():
            bf16 = jnp.bfloat16
            f32 = jnp.float32

            sa_cw[0] = out_ref[0, sup_rows(q), :HALF].astype(bf16)
            sa_ccw[0] = out_ref[0, sup_rows(q), HALF:].astype(bf16)
            for s in range(3):
                slot = s % 2
                exchange(slot, semsA, sa_cw, ra_cw, right_xy,
                         sa_ccw, ra_ccw, left_xy)
                j_cw = lax.rem(q + 7 - s, 4)
                j_ccw = lax.rem(q + 1 + s, 4)
                v_cw = (ra_cw[slot].astype(f32)
                        + out_ref[0, sup_rows(j_cw), :HALF])
                v_ccw = (ra_ccw[slot].astype(f32)
                         + out_ref[0, sup_rows(j_ccw), HALF:])
                if s < 2:
                    sa_cw[(s + 1) % 2] = v_cw.astype(bf16)
                    sa_ccw[(s + 1) % 2] = v_ccw.astype(bf16)
                else:
                    out_ref[0, sup_rows(j_cw), :HALF] = v_cw
                    out_ref[0, sup_rows(j_ccw), HALF:] = v_ccw

            row0_cw = lax.rem(q + 1, 4) * SUP
            row0_ccw = lax.rem(q + 3, 4) * SUP

            o1 = b0 * 128
            o1s = (1 - b0) * 128
            o2 = o1 + b1 * CHUNK
            o2s = o1 + (1 - b1) * CHUNK

            def pairwise(slot, nrows, dst):
                rd = pltpu.make_async_remote_copy(
                    src_ref=sb_x.at[slot, :nrows], dst_ref=rb_x.at[slot, :nrows],
                    send_sem=semsB.at[0, slot], recv_sem=semsB.at[1, slot],
                    device_id=(dst,), device_id_type=pl.DeviceIdType.MESH)
                rd.start()
                rd.wait()

            def pack(slot, nrows, off):
                sb_x[slot, :nrows, :HALF] = out_ref[
                    0, pl.ds(row0_cw + off, nrows), :HALF].astype(bf16)
                sb_x[slot, :nrows, HALF:] = out_ref[
                    0, pl.ds(row0_ccw + off, nrows), HALF:].astype(bf16)

            def unpack_add(slot, nrows, off):
                out_ref[0, pl.ds(row0_cw + off, nrows), :HALF] = (
                    out_ref[0, pl.ds(row0_cw + off, nrows), :HALF]
                    + rb_x[slot, :nrows, :HALF].astype(f32))
                out_ref[0, pl.ds(row0_ccw + off, nrows), HALF:] = (
                    out_ref[0, pl.ds(row0_ccw + off, nrows), HALF:]
                    + rb_x[slot, :nrows, HALF:].astype(f32))

            def unpack_store(slot, nrows, off):
                out_ref[0, pl.ds(row0_cw + off, nrows), :HALF] = \
                    rb_x[slot, :nrows, :HALF].astype(f32)
                out_ref[0, pl.ds(row0_ccw + off, nrows), HALF:] = \
                    rb_x[slot, :nrows, HALF:].astype(f32)

            pack(0, 128, o1s)
            pairwise(0, 128, p1)
            unpack_add(0, 128, o1)
            pack(1, CHUNK, o2s)
            pairwise(1, CHUNK, p2)
            unpack_add(1, CHUNK, o2)
            pack(2, CHUNK, o2)
            pairwise(2, CHUNK, p2)
            unpack_store(2, CHUNK, o2s)
            pack(3, 128, o1)
            pairwise(3, 128, p1)
            unpack_store(3, 128, o1s)

            sc_cw[0] = out_ref[0, sup_rows(lax.rem(q + 1, 4)),
                               :HALF].astype(bf16)
            sc_ccw[0] = out_ref[0, sup_rows(lax.rem(q + 3, 4)),
                                HALF:].astype(bf16)
            for g in range(3):
                slot = g % 2
                exchange(slot, semsC, sc_cw, rc_cw, right_xy,
                         sc_ccw, rc_ccw, left_xy)
                j_cw = lax.rem(q + 4 - g, 4)
                j_ccw = lax.rem(q + g, 4)
                out_ref[0, sup_rows(j_cw), :HALF] = rc_cw[slot].astype(f32)
                out_ref[0, sup_rows(j_ccw), HALF:] = rc_ccw[slot].astype(f32)
                if g < 2:
                    sc_cw[(g + 1) % 2] = rc_cw[slot]
                    sc_ccw[(g + 1) % 2] = rc_ccw[slot]

        cp_wq = pltpu.make_async_copy(
            wq_ref.at[:, pl.ds(my * D_MODEL, D_MODEL)], wq_s, load_sems.at[0])
        cp_wo = pltpu.make_async_copy(
            wo_ref.at[pl.ds(my * D_MODEL, D_MODEL), :], wo_s, load_sems.at[1])
        cp_wq.start()
        cp_wo.start()

        def kv_copy(h):
            s = h % 2
            ck = pltpu.make_async_copy(
                k_ref.at[0, :, h, :], k_s.at[s], load_sems.at[2 + 2 * s])
            cv = pltpu.make_async_copy(
                v_ref.at[0, :, h, :], v_s.at[s], load_sems.at[3 + 2 * s])
            return ck, cv

        cp_kv = kv_copy(0)
        cp_kv[0].start()
        cp_kv[1].start()

        barrier_sem = pltpu.get_barrier_semaphore()
        for nbr in (left_xy, right_xy, p1, p2):
            pl.semaphore_signal(
                barrier_sem, inc=1,
                device_id=(nbr,), device_id_type=pl.DeviceIdType.MESH,
            )
        pl.semaphore_wait(barrier_sem, 4)

        if _ABLATE == 'nocomp':
            cp_wq.wait()
            cp_wo.wait()
            cp_kv[0].wait()
            cp_kv[1].wait()
            out_ref[0] = x_ref[0]
            run_allreduce()
            return
        cp_wq.wait()
        xb = x_ref[0].astype(jnp.bfloat16)
        wqb = wq_s[...].astype(jnp.bfloat16)
        q_all = jnp.dot(xb, wqb, preferred_element_type=jnp.float32)

        qb = lax.broadcasted_iota(jnp.int32, (SQ, SKV), 0) // 64
        kb = lax.broadcasted_iota(jnp.int32, (SQ, SKV), 1) // 64
        mask = (qb == kb) | (kb == 0) | (lax.rem(qb + kb, 3) == 0)
        bias = jnp.where(mask, 0.0, -1e9).astype(jnp.float32)

        if _ABLATE == 'noattn':
            cp_kv[0].wait()
            cp_kv[1].wait()
            ctx_s[...] = q_all.astype(jnp.bfloat16)
        else:
            for h in range(H_LOCAL):
                ck, cv = cp_kv
                ck.wait()
                cv.wait()
                if h + 1 < H_LOCAL:
                    cp_kv = kv_copy(h + 1)
                    cp_kv[0].start()
                    cp_kv[1].start()
                q_h = q_all[:, h * DH:(h + 1) * DH].astype(jnp.bfloat16)
                k_h = k_s[h % 2].astype(jnp.bfloat16)
                v_h = v_s[h % 2].astype(jnp.bfloat16)
                scores = lax.dot_general(
                    q_h, k_h, (((1,), (1,)), ((), ())),
                    preferred_element_type=jnp.float32,
                ) * SCALE + bias
                if _ABLATE == 'nosoftmax':
                    w = scores
                else:
                    e = jnp.exp(scores)
                    recip = 1.0 / jnp.sum(e, axis=1, keepdims=True)
                    w = e * recip
                ctx_h = jnp.dot(w.astype(jnp.bfloat16), v_h,
                                preferred_element_type=jnp.float32)
                ctx_s[:, h * DH:(h + 1) * DH] = ctx_h.astype(jnp.bfloat16)

        cp_wo.wait()
        wob = wo_s[...].astype(jnp.bfloat16)
        out_ref[0] = jnp.dot(ctx_s[...], wob,
                             preferred_element_type=jnp.float32)

        if _ABLATE != 'noring':
            run_allreduce()

    return pl.pallas_call(
        body,
        out_shape=jax.ShapeDtypeStruct((1, SQ, D_MODEL), jnp.float32),
        in_specs=[
            pl.BlockSpec(memory_space=pltpu.VMEM),
            pl.BlockSpec(memory_space=pl.ANY),
            pl.BlockSpec(memory_space=pl.ANY),
            pl.BlockSpec(memory_space=pl.ANY),
            pl.BlockSpec(memory_space=pl.ANY),
        ],
        out_specs=pl.BlockSpec(memory_space=pltpu.VMEM),
        scratch_shapes=[
            pltpu.VMEM((D_MODEL, D_MODEL), jnp.float32),
            pltpu.VMEM((D_MODEL, D_MODEL), jnp.float32),
            pltpu.VMEM((2, SKV, DH), jnp.float32),
            pltpu.VMEM((2, SKV, DH), jnp.float32),
            pltpu.VMEM((SQ, D_MODEL), jnp.bfloat16),
            pltpu.VMEM((2, SUP, HALF), jnp.bfloat16),
            pltpu.VMEM((2, SUP, HALF), jnp.bfloat16),
            pltpu.VMEM((2, SUP, HALF), jnp.bfloat16),
            pltpu.VMEM((2, SUP, HALF), jnp.bfloat16),
            pltpu.VMEM((4, 128, D_MODEL), jnp.bfloat16),
            pltpu.VMEM((4, 128, D_MODEL), jnp.bfloat16),
            pltpu.VMEM((2, SUP, HALF), jnp.bfloat16),
            pltpu.VMEM((2, SUP, HALF), jnp.bfloat16),
            pltpu.VMEM((2, SUP, HALF), jnp.bfloat16),
            pltpu.VMEM((2, SUP, HALF), jnp.bfloat16),
            pltpu.SemaphoreType.DMA((4, 2)),
            pltpu.SemaphoreType.DMA((2, 4)),
            pltpu.SemaphoreType.DMA((4, 2)),
            pltpu.SemaphoreType.DMA((6,)),
        ],
        compiler_params=pltpu.CompilerParams(collective_id=0),
    )(x, Wq, K_ext, V_ext, Wo)


# device time: 89885 ns/iter; 2.0291x vs baseline; 1.0204x over previous
import os

import jax
import jax.numpy as jnp
from jax import lax
from jax.experimental import pallas as pl
from jax.experimental.pallas import tpu as pltpu

_ABLATE = os.environ.get('KERNEL_ABLATE', '')

N_DEV = 16
SQ = 1024
SKV = 1024
H_LOCAL = 8
DH = 128
D_MODEL = 1024
HALF = D_MODEL // 2
CHUNK = 64
SUP = 256
SCALE = 0.08838834764831843

GRAY = (0, 1, 3, 2)


def kernel(x, Wq, K_ext, V_ext, Wo):
    def body(x_ref, wq_ref, k_ref, v_ref, wo_ref, out_ref,
             wq_s, wo_s, k_s, v_s, ctx_s,
             sa_cw, ra_cw, sa_ccw, ra_ccw,
             sb_x, rb_x,
             sc_cw, rc_cw, sc_ccw, rc_ccw,
             semsA, semsB, semsC, load_sems):
        my = lax.axis_index("i")
        q = lax.rem(my, 4)
        z = my // 4
        right_xy = 4 * z + lax.rem(q + 1, 4)
        left_xy = 4 * z + lax.rem(q + 3, 4)
        b0 = lax.rem(z, 2)
        b1 = lax.rem(z // 2, 2)
        p1 = 4 * (z + 1 - 2 * b0) + q
        p2 = 4 * (z + 2 - 4 * b1) + q

        def exchange_start(slot, sems, s_cw, r_cw, dst_cw, s_ccw, r_ccw,
                           dst_ccw):
            r1 = pltpu.make_async_remote_copy(
                src_ref=s_cw.at[slot], dst_ref=r_cw.at[slot],
                send_sem=sems.at[0, slot], recv_sem=sems.at[1, slot],
                device_id=(dst_cw,), device_id_type=pl.DeviceIdType.MESH)
            r2 = pltpu.make_async_remote_copy(
                src_ref=s_ccw.at[slot], dst_ref=r_ccw.at[slot],
                send_sem=sems.at[2, slot], recv_sem=sems.at[3, slot],
                device_id=(dst_ccw,), device_id_type=pl.DeviceIdType.MESH)
            r1.start()
            r2.start()
            return r1, r2

        def exchange(slot, sems, s_cw, r_cw, dst_cw, s_ccw, r_ccw, dst_ccw):
            r1, r2 = exchange_start(slot, sems, s_cw, r_cw, dst_cw,
                                    s_ccw, r_ccw, dst_ccw)
            r1.wait()
            r2.wait()

        def sup_rows(j):
            return pl.ds(j * SUP, SUP)

        def run_allreduce(phalf=None):
            bf16 = jnp.bfloat16
            f32 = jnp.float32

            if phalf is None:
                def phalf(j, lo, hi):
                    return out_ref[0, sup_rows(j), lo:hi]

            sa_cw[0] = phalf(q, 0, HALF).astype(bf16)
            sa_ccw[0] = phalf(q, HALF, D_MODEL).astype(bf16)
            for s in range(3):
                slot = s % 2
                r1, r2 = exchange_start(slot, semsA, sa_cw, ra_cw, right_xy,
                                        sa_ccw, ra_ccw, left_xy)
                j_cw = lax.rem(q + 7 - s, 4)
                j_ccw = lax.rem(q + 1 + s, 4)
                p_cw = phalf(j_cw, 0, HALF)
                p_ccw = phalf(j_ccw, HALF, D_MODEL)
                r1.wait()
                r2.wait()
                v_cw = ra_cw[slot].astype(f32) + p_cw
                v_ccw = ra_ccw[slot].astype(f32) + p_ccw
                if s < 2:
                    sa_cw[(s + 1) % 2] = v_cw.astype(bf16)
                    sa_ccw[(s + 1) % 2] = v_ccw.astype(bf16)
                else:
                    out_ref[0, sup_rows(j_cw), :HALF] = v_cw
                    out_ref[0, sup_rows(j_ccw), HALF:] = v_ccw

            row0_cw = lax.rem(q + 1, 4) * SUP
            row0_ccw = lax.rem(q + 3, 4) * SUP

            o1 = b0 * 128
            o1s = (1 - b0) * 128
            o2 = o1 + b1 * CHUNK
            o2s = o1 + (1 - b1) * CHUNK

            def pairwise(slot, nrows, dst):
                rd = pltpu.make_async_remote_copy(
                    src_ref=sb_x.at[slot, :nrows], dst_ref=rb_x.at[slot, :nrows],
                    send_sem=semsB.at[0, slot], recv_sem=semsB.at[1, slot],
                    device_id=(dst,), device_id_type=pl.DeviceIdType.MESH)
                rd.start()
                rd.wait()

            def pack(slot, nrows, off):
                sb_x[slot, :nrows, :HALF] = out_ref[
                    0, pl.ds(row0_cw + off, nrows), :HALF].astype(bf16)
                sb_x[slot, :nrows, HALF:] = out_ref[
                    0, pl.ds(row0_ccw + off, nrows), HALF:].astype(bf16)

            def unpack_add(slot, nrows, off):
                out_ref[0, pl.ds(row0_cw + off, nrows), :HALF] = (
                    out_ref[0, pl.ds(row0_cw + off, nrows), :HALF]
                    + rb_x[slot, :nrows, :HALF].astype(f32))
                out_ref[0, pl.ds(row0_ccw + off, nrows), HALF:] = (
                    out_ref[0, pl.ds(row0_ccw + off, nrows), HALF:]
                    + rb_x[slot, :nrows, HALF:].astype(f32))

            def unpack_store(slot, nrows, off):
                out_ref[0, pl.ds(row0_cw + off, nrows), :HALF] = \
                    rb_x[slot, :nrows, :HALF].astype(f32)
                out_ref[0, pl.ds(row0_ccw + off, nrows), HALF:] = \
                    rb_x[slot, :nrows, HALF:].astype(f32)

            pack(0, 128, o1s)
            pairwise(0, 128, p1)
            unpack_add(0, 128, o1)
            pack(1, CHUNK, o2s)
            pairwise(1, CHUNK, p2)
            unpack_add(1, CHUNK, o2)
            pack(2, CHUNK, o2)
            pairwise(2, CHUNK, p2)
            unpack_store(2, CHUNK, o2s)
            pack(3, 128, o1)
            pairwise(3, 128, p1)
            unpack_store(3, 128, o1s)

            sc_cw[0] = out_ref[0, sup_rows(lax.rem(q + 1, 4)),
                               :HALF].astype(bf16)
            sc_ccw[0] = out_ref[0, sup_rows(lax.rem(q + 3, 4)),
                                HALF:].astype(bf16)
            for g in range(3):
                slot = g % 2
                exchange(slot, semsC, sc_cw, rc_cw, right_xy,
                         sc_ccw, rc_ccw, left_xy)
                j_cw = lax.rem(q + 4 - g, 4)
                j_ccw = lax.rem(q + g, 4)
                out_ref[0, sup_rows(j_cw), :HALF] = rc_cw[slot].astype(f32)
                out_ref[0, sup_rows(j_ccw), HALF:] = rc_ccw[slot].astype(f32)
                if g < 2:
                    sc_cw[(g + 1) % 2] = rc_cw[slot]
                    sc_ccw[(g + 1) % 2] = rc_ccw[slot]

        cp_wq = pltpu.make_async_copy(
            wq_ref.at[:, pl.ds(my * D_MODEL, D_MODEL)], wq_s, load_sems.at[0])
        cp_wo = pltpu.make_async_copy(
            wo_ref.at[pl.ds(my * D_MODEL, D_MODEL), :], wo_s, load_sems.at[1])
        cp_wq.start()
        cp_wo.start()

        def kv_copy(h):
            s = h % 2
            ck = pltpu.make_async_copy(
                k_ref.at[0, :, h, :], k_s.at[s], load_sems.at[2 + 2 * s])
            cv = pltpu.make_async_copy(
                v_ref.at[0, :, h, :], v_s.at[s], load_sems.at[3 + 2 * s])
            return ck, cv

        cp_kv = kv_copy(0)
        cp_kv[0].start()
        cp_kv[1].start()

        barrier_sem = pltpu.get_barrier_semaphore()
        for nbr in (left_xy, right_xy, p1, p2):
            pl.semaphore_signal(
                barrier_sem, inc=1,
                device_id=(nbr,), device_id_type=pl.DeviceIdType.MESH,
            )
        pl.semaphore_wait(barrier_sem, 4)

        if _ABLATE == 'nocomp':
            cp_wq.wait()
            cp_wo.wait()
            cp_kv[0].wait()
            cp_kv[1].wait()
            out_ref[0] = x_ref[0]
            run_allreduce()
            return
        cp_wq.wait()
        xb = x_ref[0].astype(jnp.bfloat16)
        wqb = wq_s[...].astype(jnp.bfloat16)
        q_all = jnp.dot(xb, wqb, preferred_element_type=jnp.float32) * SCALE

        qb = lax.broadcasted_iota(jnp.int32, (SQ, SKV), 0) // 64
        kb = lax.broadcasted_iota(jnp.int32, (SQ, SKV), 1) // 64
        mask = (qb == kb) | (kb == 0) | (lax.rem(qb + kb, 3) == 0)
        bias = jnp.where(mask, 0.0, -1e9).astype(jnp.float32)

        if _ABLATE == 'noattn':
            cp_kv[0].wait()
            cp_kv[1].wait()
            ctx_s[...] = q_all.astype(jnp.bfloat16)
        else:
            for h in range(H_LOCAL):
                ck, cv = cp_kv
                ck.wait()
                cv.wait()
                if h + 1 < H_LOCAL:
                    cp_kv = kv_copy(h + 1)
                    cp_kv[0].start()
                    cp_kv[1].start()
                q_h = q_all[:, h * DH:(h + 1) * DH].astype(jnp.bfloat16)
                k_h = k_s[h % 2].astype(jnp.bfloat16)
                v_h = v_s[h % 2].astype(jnp.bfloat16)
                scores = lax.dot_general(
                    q_h, k_h, (((1,), (1,)), ((), ())),
                    preferred_element_type=jnp.float32,
                ) + bias
                if _ABLATE == 'nosoftmax':
                    w = scores
                else:
                    e = jnp.exp(scores)
                    recip = 1.0 / jnp.sum(e, axis=1, keepdims=True)
                    w = e * recip
                ctx_h = jnp.dot(w.astype(jnp.bfloat16), v_h,
                                preferred_element_type=jnp.float32)
                ctx_s[:, h * DH:(h + 1) * DH] = ctx_h.astype(jnp.bfloat16)

        cp_wo.wait()
        wob = wo_s[...].astype(jnp.bfloat16)

        def phalf(j, lo, hi):
            return jnp.dot(ctx_s[pl.ds(j * SUP, SUP), :], wob[:, lo:hi],
                           preferred_element_type=jnp.float32)

        if _ABLATE == 'noring':
            out_ref[0] = jnp.dot(ctx_s[...], wob,
                                 preferred_element_type=jnp.float32)
        else:
            run_allreduce(phalf)

    return pl.pallas_call(
        body,
        out_shape=jax.ShapeDtypeStruct((1, SQ, D_MODEL), jnp.float32),
        in_specs=[
            pl.BlockSpec(memory_space=pltpu.VMEM),
            pl.BlockSpec(memory_space=pl.ANY),
            pl.BlockSpec(memory_space=pl.ANY),
            pl.BlockSpec(memory_space=pl.ANY),
            pl.BlockSpec(memory_space=pl.ANY),
        ],
        out_specs=pl.BlockSpec(memory_space=pltpu.VMEM),
        scratch_shapes=[
            pltpu.VMEM((D_MODEL, D_MODEL), jnp.float32),
            pltpu.VMEM((D_MODEL, D_MODEL), jnp.float32),
            pltpu.VMEM((2, SKV, DH), jnp.float32),
            pltpu.VMEM((2, SKV, DH), jnp.float32),
            pltpu.VMEM((SQ, D_MODEL), jnp.bfloat16),
            pltpu.VMEM((2, SUP, HALF), jnp.bfloat16),
            pltpu.VMEM((2, SUP, HALF), jnp.bfloat16),
            pltpu.VMEM((2, SUP, HALF), jnp.bfloat16),
            pltpu.VMEM((2, SUP, HALF), jnp.bfloat16),
            pltpu.VMEM((4, 128, D_MODEL), jnp.bfloat16),
            pltpu.VMEM((4, 128, D_MODEL), jnp.bfloat16),
            pltpu.VMEM((2, SUP, HALF), jnp.bfloat16),
            pltpu.VMEM((2, SUP, HALF), jnp.bfloat16),
            pltpu.VMEM((2, SUP, HALF), jnp.bfloat16),
            pltpu.VMEM((2, SUP, HALF), jnp.bfloat16),
            pltpu.SemaphoreType.DMA((4, 2)),
            pltpu.SemaphoreType.DMA((2, 4)),
            pltpu.SemaphoreType.DMA((4, 2)),
            pltpu.SemaphoreType.DMA((6,)),
        ],
        compiler_params=pltpu.CompilerParams(collective_id=0),
    )(x, Wq, K_ext, V_ext, Wo)
